# Initial kernel scaffold; baseline (speedup 1.0000x reference)
import jax
import jax.numpy as jnp
from jax.experimental import pallas as pl


def kernel(features, W0, W1):
    raise NotImplementedError("write your pallas kernel here")


# speedup vs baseline: 14.5857x; 14.5857x over previous
import jax
import jax.numpy as jnp
from jax.experimental import pallas as pl

_K = 31
_NEG = -3.0e38
_POS = 3.0e38


def _emb_kernel(f_ref, w0_ref, w1_ref, o_ref):
    h = jnp.maximum(f_ref[...] * w0_ref[...], 0.0) * w1_ref[...]
    n = jnp.sqrt(jnp.sum(h * h, axis=1, keepdims=True))
    o_ref[...] = h / jnp.maximum(n, 1e-12)


def _topk_kernel(embs_ref, emba_ref, o_ref):
    s = jax.lax.dot_general(
        embs_ref[...], emba_ref[...],
        (((1,), (1,)), ((), ())),
        preferred_element_type=jnp.float32,
    )

    def body(_, cur):
        m = jnp.where(s < cur, s, _NEG)
        return jnp.max(m, axis=1, keepdims=True)

    t = jax.lax.fori_loop(
        0, _K, body, jnp.full((s.shape[0], 1), _POS, dtype=jnp.float32))
    o_ref[...] = jnp.where(s >= t, jnp.maximum(s, 0.0), 0.0)


def kernel(features, W0, W1):
    n, d = features.shape
    w0 = W0.reshape(1, d).astype(jnp.float32)
    w1 = W1.reshape(1, d).astype(jnp.float32)
    emb = pl.pallas_call(
        _emb_kernel,
        out_shape=jax.ShapeDtypeStruct((n, d), jnp.float32),
    )(features.astype(jnp.float32), w0, w1)

    r = 400 if n % 400 == 0 else n
    out = pl.pallas_call(
        _topk_kernel,
        grid=(n // r,),
        in_specs=[
            pl.BlockSpec((r, d), lambda i: (i, 0)),
            pl.BlockSpec((n, d), lambda i: (0, 0)),
        ],
        out_specs=pl.BlockSpec((r, n), lambda i: (i, 0)),
        out_shape=jax.ShapeDtypeStruct((n, n), jnp.float32),
    )(emb, emb)
    return out


# speedup vs baseline: 14.5869x; 1.0001x over previous
import jax
import jax.numpy as jnp
from jax.experimental import pallas as pl
from jax.experimental.pallas import tpu as pltpu

_K = 31
_NEG = -3.0e38
_POS = 3.0e38


def _emb_kernel(f_ref, w0_ref, w1_ref, o_ref):
    h = jnp.maximum(f_ref[...] * w0_ref[...], 0.0) * w1_ref[...]
    n = jnp.sqrt(jnp.sum(h * h, axis=1, keepdims=True))
    o_ref[...] = h / jnp.maximum(n, 1e-12)


def _topk_kernel(embs_ref, emba_ref, o_ref):
    s = jax.lax.dot_general(
        embs_ref[...], emba_ref[...],
        (((1,), (1,)), ((), ())),
        preferred_element_type=jnp.float32,
    )

    def body(_, cur):
        m = jnp.where(s < cur, s, _NEG)
        return jnp.max(m, axis=1, keepdims=True)

    t = jax.lax.fori_loop(
        0, _K, body, jnp.full((s.shape[0], 1), _POS, dtype=jnp.float32))
    o_ref[...] = jnp.where(s >= t, jnp.maximum(s, 0.0), 0.0)


def kernel(features, W0, W1):
    n, d = features.shape
    w0 = W0.reshape(1, d).astype(jnp.float32)
    w1 = W1.reshape(1, d).astype(jnp.float32)
    emb = pl.pallas_call(
        _emb_kernel,
        out_shape=jax.ShapeDtypeStruct((n, d), jnp.float32),
    )(features.astype(jnp.float32), w0, w1)

    r = 400 if n % 400 == 0 else n
    out = pl.pallas_call(
        _topk_kernel,
        grid=(n // r,),
        in_specs=[
            pl.BlockSpec((r, d), lambda i: (i, 0)),
            pl.BlockSpec((n, d), lambda i: (0, 0)),
        ],
        out_specs=pl.BlockSpec((r, n), lambda i: (i, 0)),
        out_shape=jax.ShapeDtypeStruct((n, n), jnp.float32),
        compiler_params=pltpu.CompilerParams(
            dimension_semantics=("parallel",)),
    )(emb, emb)
    return out


# speedup vs baseline: 29.1345x; 1.9973x over previous
import jax
import jax.numpy as jnp
from jax.experimental import pallas as pl
from jax.experimental.pallas import tpu as pltpu

_K = 31
_NEG = -3.0e38
_POS = 3.0e38
_NPAD = 10240


def _emb_kernel(f_ref, w0_ref, w1_ref, o_ref):
    h = jnp.maximum(f_ref[...] * w0_ref[...], 0.0) * w1_ref[...]
    n = jnp.sqrt(jnp.sum(h * h, axis=1, keepdims=True))
    o_ref[...] = h / jnp.maximum(n, 1e-12)


def _swap(a, b):
    return jnp.maximum(a, b), jnp.minimum(a, b)


def _sorted4(a0, a1, b0, b1):
    c0, t = _swap(a0, b0)
    u, c3 = _swap(a1, b1)
    c1, c2 = _swap(t, u)
    return [c0, c1, c2, c3]


def _quads(xs):
    hi, lo = [], []
    for i in range(8):
        h, l = _swap(xs[2 * i], xs[2 * i + 1])
        hi.append(h)
        lo.append(l)
    return [_sorted4(hi[2 * q], lo[2 * q], hi[2 * q + 1], lo[2 * q + 1])
            for q in range(4)]


def _merge44_top4(a, b):
    d = [jnp.maximum(a[i], b[3 - i]) for i in range(4)]
    e0, e2 = _swap(d[0], d[2])
    e1, e3 = _swap(d[1], d[3])
    f0, f1 = _swap(e0, e1)
    f2, f3 = _swap(e2, e3)
    return [f0, f1, f2, f3]


def _top4_of16(xs):
    q = _quads(xs)
    s01 = _merge44_top4(q[0], q[1])
    s23 = _merge44_top4(q[2], q[3])
    return [jnp.maximum(s01[i], s23[3 - i]) for i in range(4)]


def _bitonic8_desc(v):
    v = list(v)
    for dist in (4, 2, 1):
        nv = list(v)
        for i in range(8):
            j = i + dist
            if j < 8 and (i // dist) % 2 == 0:
                nv[i], nv[j] = _swap(v[i], v[j])
        v = nv
    return v


def _merge44_sorted8(a, b):
    return _bitonic8_desc(a + b[::-1])


def _merge88_top8(a, b):
    d = [jnp.maximum(a[i], b[7 - i]) for i in range(8)]
    return _bitonic8_desc(d)


def _top_m(xs, m):
    neg = jnp.full_like(xs[0], _NEG)
    xs = list(xs)
    while len(xs) % 4:
        xs.append(neg)
    quads = []
    for i in range(0, len(xs), 4):
        h0, l0 = _swap(xs[i], xs[i + 1])
        h1, l1 = _swap(xs[i + 2], xs[i + 3])
        quads.append(_sorted4(h0, l0, h1, l1))
    s8s = []
    for i in range(0, len(quads) - 1, 2):
        s8s.append(_merge44_sorted8(quads[i], quads[i + 1]))
    if len(quads) % 2:
        s8s.append(quads[-1] + [neg] * 4)
    acc = s8s[0]
    for s8 in s8s[1:]:
        acc = _merge88_top8(acc, s8)
    return acc[:m]


def _extract_kth(x, ub):
    def body(_, cur):
        m = jnp.where(x < cur, x, _NEG)
        return jnp.max(m, axis=1, keepdims=True)
    return jax.lax.fori_loop(
        0, ub, body, jnp.full((x.shape[0], 1), _POS, dtype=jnp.float32))


def _topk_kernel(embs_ref, emba_ref, o_ref):
    s = jax.lax.dot_general(
        embs_ref[...], emba_ref[...],
        (((1,), (1,)), ((), ())),
        preferred_element_type=jnp.float32,
    )

    w = s.shape[1]
    g1 = w // 16
    xs = [s[:, i * g1:(i + 1) * g1] for i in range(16)]
    m4 = _top4_of16(xs)
    cand = jnp.concatenate(m4, axis=1)

    g2 = 128 if (4 * g1) % 128 == 0 else g1
    ys = [cand[:, i * g2:(i + 1) * g2] for i in range((4 * g1) // g2)]
    m6 = _top_m(ys, 6)
    cand2 = jnp.concatenate(m6, axis=1)

    t = _extract_kth(cand2, _K)

    cnt = jnp.sum(jnp.where(s >= t, 1.0, 0.0), axis=1, keepdims=True)
    deficit = jnp.max(cnt) > float(_K)
    ub = jnp.where(deficit, _K, 0)
    t_fb = _extract_kth(s, ub)
    t = jnp.where(cnt > float(_K), t_fb, t)

    o_ref[...] = jnp.where(s[:, :o_ref.shape[1]] >= t,
                           jnp.maximum(s[:, :o_ref.shape[1]], 0.0), 0.0)


def kernel(features, W0, W1):
    n, d = features.shape
    w0 = W0.reshape(1, d).astype(jnp.float32)
    w1 = W1.reshape(1, d).astype(jnp.float32)
    emb = pl.pallas_call(
        _emb_kernel,
        out_shape=jax.ShapeDtypeStruct((n, d), jnp.float32),
    )(features.astype(jnp.float32), w0, w1)

    npad = _NPAD if n == 10000 else n
    if npad != n:
        emba = jnp.concatenate(
            [emb, jnp.zeros((npad - n, d), jnp.float32)], axis=0)
    else:
        emba = emb

    r = 200 if n % 200 == 0 else n
    out = pl.pallas_call(
        _topk_kernel,
        grid=(n // r,),
        in_specs=[
            pl.BlockSpec((r, d), lambda i: (i, 0)),
            pl.BlockSpec((npad, d), lambda i: (0, 0)),
        ],
        out_specs=pl.BlockSpec((r, n), lambda i: (i, 0)),
        out_shape=jax.ShapeDtypeStruct((n, n), jnp.float32),
        compiler_params=pltpu.CompilerParams(
            dimension_semantics=("parallel",)),
    )(emb, emba)
    return out


# speedup vs baseline: 31.2626x; 1.0730x over previous
import jax
import jax.numpy as jnp
from jax.experimental import pallas as pl
from jax.experimental.pallas import tpu as pltpu

_K = 31
_NEG = -3.0e38
_POS = 3.0e38
_NPAD = 10240


def _emb_kernel(f_ref, w0_ref, w1_ref, o_ref):
    h = jnp.maximum(f_ref[...] * w0_ref[...], 0.0) * w1_ref[...]
    n = jnp.sqrt(jnp.sum(h * h, axis=1, keepdims=True))
    o_ref[...] = h / jnp.maximum(n, 1e-12)


def _swap(a, b):
    return jnp.maximum(a, b), jnp.minimum(a, b)


def _sorted4(a0, a1, b0, b1):
    c0, t = _swap(a0, b0)
    u, c3 = _swap(a1, b1)
    c1, c2 = _swap(t, u)
    return [c0, c1, c2, c3]


def _quads(xs):
    hi, lo = [], []
    for i in range(8):
        h, l = _swap(xs[2 * i], xs[2 * i + 1])
        hi.append(h)
        lo.append(l)
    return [_sorted4(hi[2 * q], lo[2 * q], hi[2 * q + 1], lo[2 * q + 1])
            for q in range(4)]


def _merge44_top4(a, b):
    d = [jnp.maximum(a[i], b[3 - i]) for i in range(4)]
    e0, e2 = _swap(d[0], d[2])
    e1, e3 = _swap(d[1], d[3])
    f0, f1 = _swap(e0, e1)
    f2, f3 = _swap(e2, e3)
    return [f0, f1, f2, f3]


def _top4_of16(xs):
    q = _quads(xs)
    s01 = _merge44_top4(q[0], q[1])
    s23 = _merge44_top4(q[2], q[3])
    return [jnp.maximum(s01[i], s23[3 - i]) for i in range(4)]


def _bitonic8_desc(v):
    v = list(v)
    for dist in (4, 2, 1):
        nv = list(v)
        for i in range(8):
            j = i + dist
            if j < 8 and (i // dist) % 2 == 0:
                nv[i], nv[j] = _swap(v[i], v[j])
        v = nv
    return v


def _merge44_sorted8(a, b):
    return _bitonic8_desc(a + b[::-1])


def _merge88_top8(a, b):
    d = [jnp.maximum(a[i], b[7 - i]) for i in range(8)]
    return _bitonic8_desc(d)


def _top_m(xs, m):
    neg = jnp.full_like(xs[0], _NEG)
    xs = list(xs)
    while len(xs) % 4:
        xs.append(neg)
    quads = []
    for i in range(0, len(xs), 4):
        h0, l0 = _swap(xs[i], xs[i + 1])
        h1, l1 = _swap(xs[i + 2], xs[i + 3])
        quads.append(_sorted4(h0, l0, h1, l1))
    s8s = []
    for i in range(0, len(quads) - 1, 2):
        s8s.append(_merge44_sorted8(quads[i], quads[i + 1]))
    if len(quads) % 2:
        s8s.append(quads[-1] + [neg] * 4)
    acc = s8s[0]
    for s8 in s8s[1:]:
        acc = _merge88_top8(acc, s8)
    return acc[:m]


def _extract_kth(x, ub):
    def body(_, cur):
        m = jnp.where(x < cur, x, _NEG)
        return jnp.max(m, axis=1, keepdims=True)
    return jax.lax.fori_loop(
        0, ub, body, jnp.full((x.shape[0], 1), _POS, dtype=jnp.float32))


def _sim_strip(embs_ref, emba_ref):
    return jax.lax.dot_general(
        embs_ref[...], emba_ref[...],
        (((1,), (1,)), ((), ())),
        preferred_element_type=jnp.float32,
    )


def _cand_kernel(embs_ref, emba_ref, c_ref):
    s = _sim_strip(embs_ref, emba_ref)
    w = s.shape[1]
    g1 = w // 16
    xs = [s[:, i * g1:(i + 1) * g1] for i in range(16)]
    m4 = _top4_of16(xs)
    cand = jnp.concatenate(m4, axis=1)
    g2 = 128 if (4 * g1) % 128 == 0 else g1
    ys = [cand[:, i * g2:(i + 1) * g2] for i in range((4 * g1) // g2)]
    m6 = _top_m(ys, 6)
    c_ref[...] = jnp.concatenate(m6, axis=1)


def _thresh_kernel(c_ref, t_ref):
    t = _extract_kth(c_ref[...], _K)
    t_ref[...] = jnp.broadcast_to(t, t_ref.shape)


def _mask_kernel(embs_ref, emba_ref, t_ref, o_ref):
    s = _sim_strip(embs_ref, emba_ref)
    no = o_ref.shape[1]
    t = t_ref[...][:, 0:1]
    msk = s >= t
    o_ref[...] = jnp.where(msk[:, :no], jnp.maximum(s[:, :no], 0.0), 0.0)
    cnt = jnp.sum(jnp.where(msk, 1.0, 0.0), axis=1, keepdims=True)

    @pl.when(jnp.max(cnt) > float(_K))
    def _fallback():
        t2 = _extract_kth(s, _K)
        o_ref[...] = jnp.where((s >= t2)[:, :no],
                               jnp.maximum(s[:, :no], 0.0), 0.0)


def kernel(features, W0, W1):
    n, d = features.shape
    w0 = W0.reshape(1, d).astype(jnp.float32)
    w1 = W1.reshape(1, d).astype(jnp.float32)
    emb = pl.pallas_call(
        _emb_kernel,
        out_shape=jax.ShapeDtypeStruct((n, d), jnp.float32),
    )(features.astype(jnp.float32), w0, w1)

    npad = _NPAD if n == 10000 else n
    if npad != n:
        emba = jnp.concatenate(
            [emb, jnp.zeros((npad - n, d), jnp.float32)], axis=0)
    else:
        emba = emb

    nc = 6 * (128 if (npad // 4) % 128 == 0 else npad // 16)
    ra = 400 if n % 400 == 0 else n
    cand = pl.pallas_call(
        _cand_kernel,
        grid=(n // ra,),
        in_specs=[
            pl.BlockSpec((ra, d), lambda i: (i, 0)),
            pl.BlockSpec((npad, d), lambda i: (0, 0)),
        ],
        out_specs=pl.BlockSpec((ra, nc), lambda i: (i, 0)),
        out_shape=jax.ShapeDtypeStruct((n, nc), jnp.float32),
        compiler_params=pltpu.CompilerParams(
            dimension_semantics=("parallel",)),
    )(emb, emba)

    rb = 5000 if n % 5000 == 0 else n
    thr = pl.pallas_call(
        _thresh_kernel,
        grid=(n // rb,),
        in_specs=[pl.BlockSpec((rb, nc), lambda i: (i, 0))],
        out_specs=pl.BlockSpec((rb, 128), lambda i: (i, 0)),
        out_shape=jax.ShapeDtypeStruct((n, 128), jnp.float32),
        compiler_params=pltpu.CompilerParams(
            dimension_semantics=("parallel",)),
    )(cand)

    rc = 200 if n % 200 == 0 else n
    out = pl.pallas_call(
        _mask_kernel,
        grid=(n // rc,),
        in_specs=[
            pl.BlockSpec((rc, d), lambda i: (i, 0)),
            pl.BlockSpec((npad, d), lambda i: (0, 0)),
            pl.BlockSpec((rc, 128), lambda i: (i, 0)),
        ],
        out_specs=pl.BlockSpec((rc, n), lambda i: (i, 0)),
        out_shape=jax.ShapeDtypeStruct((n, n), jnp.float32),
        compiler_params=pltpu.CompilerParams(
            dimension_semantics=("parallel",)),
    )(emb, emba, thr)
    return out


# speedup vs baseline: 46.6067x; 1.4908x over previous
import jax
import jax.numpy as jnp
from jax.experimental import pallas as pl
from jax.experimental.pallas import tpu as pltpu

_K = 31
_NEG = -3.0e38
_POS = 3.0e38
_NPAD = 10240


def _emb_kernel(f_ref, w0_ref, w1_ref, o_ref):
    h = jnp.maximum(f_ref[...] * w0_ref[...], 0.0) * w1_ref[...]
    n = jnp.sqrt(jnp.sum(h * h, axis=1, keepdims=True))
    o_ref[...] = h / jnp.maximum(n, 1e-12)


def _swap(a, b):
    return jnp.maximum(a, b), jnp.minimum(a, b)


def _sorted4(a0, a1, b0, b1):
    c0, t = _swap(a0, b0)
    u, c3 = _swap(a1, b1)
    c1, c2 = _swap(t, u)
    return [c0, c1, c2, c3]


def _quads(xs):
    hi, lo = [], []
    for i in range(8):
        h, l = _swap(xs[2 * i], xs[2 * i + 1])
        hi.append(h)
        lo.append(l)
    return [_sorted4(hi[2 * q], lo[2 * q], hi[2 * q + 1], lo[2 * q + 1])
            for q in range(4)]


def _merge44_top4(a, b):
    d = [jnp.maximum(a[i], b[3 - i]) for i in range(4)]
    e0, e2 = _swap(d[0], d[2])
    e1, e3 = _swap(d[1], d[3])
    f0, f1 = _swap(e0, e1)
    f2, f3 = _swap(e2, e3)
    return [f0, f1, f2, f3]


def _top4_of16(xs):
    q = _quads(xs)
    s01 = _merge44_top4(q[0], q[1])
    s23 = _merge44_top4(q[2], q[3])
    return [jnp.maximum(s01[i], s23[3 - i]) for i in range(4)]


def _bitonic8_desc(v):
    v = list(v)
    for dist in (4, 2, 1):
        nv = list(v)
        for i in range(8):
            j = i + dist
            if j < 8 and (i // dist) % 2 == 0:
                nv[i], nv[j] = _swap(v[i], v[j])
        v = nv
    return v


def _merge44_sorted8(a, b):
    return _bitonic8_desc(a + b[::-1])


def _merge88_top8(a, b):
    d = [jnp.maximum(a[i], b[7 - i]) for i in range(8)]
    return _bitonic8_desc(d)


def _top_m(xs, m):
    neg = jnp.full_like(xs[0], _NEG)
    xs = list(xs)
    while len(xs) % 4:
        xs.append(neg)
    quads = []
    for i in range(0, len(xs), 4):
        h0, l0 = _swap(xs[i], xs[i + 1])
        h1, l1 = _swap(xs[i + 2], xs[i + 3])
        quads.append(_sorted4(h0, l0, h1, l1))
    s8s = []
    for i in range(0, len(quads) - 1, 2):
        s8s.append(_merge44_sorted8(quads[i], quads[i + 1]))
    if len(quads) % 2:
        s8s.append(quads[-1] + [neg] * 4)
    acc = s8s[0]
    for s8 in s8s[1:]:
        acc = _merge88_top8(acc, s8)
    return acc[:m]


def _extract_kth(x, ub):
    def body(_, cur):
        m = jnp.where(x < cur, x, _NEG)
        return jnp.max(m, axis=1, keepdims=True)
    return jax.lax.fori_loop(
        0, ub, body, jnp.full((x.shape[0], 1), _POS, dtype=jnp.float32))


def _sim_strip(embs_ref, emba_ref):
    return jax.lax.dot_general(
        embs_ref[...], emba_ref[...],
        (((1,), (1,)), ((), ())),
        preferred_element_type=jnp.float32,
    )


def _cand_kernel(embs_ref, emba_ref, c_ref):
    s = _sim_strip(embs_ref, emba_ref)
    w = s.shape[1]
    g1 = w // 16
    xs = [s[:, i * g1:(i + 1) * g1] for i in range(16)]
    m4 = _top4_of16(xs)
    cand = jnp.concatenate(m4, axis=1)
    g2 = 128 if (4 * g1) % 128 == 0 else g1
    ys = [cand[:, i * g2:(i + 1) * g2] for i in range((4 * g1) // g2)]
    m6 = _top_m(ys, 6)
    c_ref[...] = jnp.concatenate(m6, axis=1)


def _thresh_kernel(c_ref, t_ref, c3_ref):
    x = c_ref[...]
    w = x.shape[1]
    if w % (6 * 128) == 0:
        zs = [x[:, j * 128 + q * 32:j * 128 + q * 32 + 32]
              for j in range(w // 128) for q in range(4)]
        x = jnp.concatenate(_top_m(zs, 8), axis=1)
    t = _extract_kth(x, _K)
    t_ref[...] = jnp.broadcast_to(t, t_ref.shape)
    c3 = jnp.sum(jnp.where(x >= t, 1.0, 0.0), axis=1, keepdims=True)
    c3_ref[...] = jnp.broadcast_to(c3, c3_ref.shape)


def _mask_kernel(embs_ref, emba_ref, t_ref, c3_ref, o_ref):
    s = _sim_strip(embs_ref, emba_ref)
    no = o_ref.shape[1]
    t = t_ref[...][:, 0:1]
    c3 = c3_ref[...][:, 0:1]
    msk = s >= t
    o_ref[...] = jnp.where(msk[:, :no], jnp.maximum(s[:, :no], 0.0), 0.0)
    cnt = jnp.sum(jnp.where(msk, 1.0, 0.0), axis=1, keepdims=True)

    @pl.when(jnp.max(jnp.abs(cnt - c3)) > 0.5)
    def _fallback():
        t2 = _extract_kth(s, _K)
        o_ref[...] = jnp.where((s >= t2)[:, :no],
                               jnp.maximum(s[:, :no], 0.0), 0.0)


def kernel(features, W0, W1):
    n, d = features.shape
    w0 = W0.reshape(1, d).astype(jnp.float32)
    w1 = W1.reshape(1, d).astype(jnp.float32)
    emb = pl.pallas_call(
        _emb_kernel,
        out_shape=jax.ShapeDtypeStruct((n, d), jnp.float32),
    )(features.astype(jnp.float32), w0, w1)

    npad = _NPAD if n == 10000 else n
    if npad != n:
        emba = jnp.concatenate(
            [emb, jnp.zeros((npad - n, d), jnp.float32)], axis=0)
    else:
        emba = emb

    nc = 6 * (128 if (npad // 4) % 128 == 0 else npad // 16)
    ra = 400 if n % 400 == 0 else n
    cand = pl.pallas_call(
        _cand_kernel,
        grid=(n // ra,),
        in_specs=[
            pl.BlockSpec((ra, d), lambda i: (i, 0)),
            pl.BlockSpec((npad, d), lambda i: (0, 0)),
        ],
        out_specs=pl.BlockSpec((ra, nc), lambda i: (i, 0)),
        out_shape=jax.ShapeDtypeStruct((n, nc), jnp.float32),
        compiler_params=pltpu.CompilerParams(
            dimension_semantics=("parallel",)),
    )(emb, emba)

    rb = 2000 if n % 2000 == 0 else n
    thr, c3 = pl.pallas_call(
        _thresh_kernel,
        grid=(n // rb,),
        in_specs=[pl.BlockSpec((rb, nc), lambda i: (i, 0))],
        out_specs=[pl.BlockSpec((rb, 128), lambda i: (i, 0)),
                   pl.BlockSpec((rb, 128), lambda i: (i, 0))],
        out_shape=[jax.ShapeDtypeStruct((n, 128), jnp.float32),
                   jax.ShapeDtypeStruct((n, 128), jnp.float32)],
        compiler_params=pltpu.CompilerParams(
            dimension_semantics=("parallel",)),
    )(cand)

    rc = 200 if n % 200 == 0 else n
    out = pl.pallas_call(
        _mask_kernel,
        grid=(n // rc,),
        in_specs=[
            pl.BlockSpec((rc, d), lambda i: (i, 0)),
            pl.BlockSpec((npad, d), lambda i: (0, 0)),
            pl.BlockSpec((rc, 128), lambda i: (i, 0)),
            pl.BlockSpec((rc, 128), lambda i: (i, 0)),
        ],
        out_specs=pl.BlockSpec((rc, n), lambda i: (i, 0)),
        out_shape=jax.ShapeDtypeStruct((n, n), jnp.float32),
        compiler_params=pltpu.CompilerParams(
            dimension_semantics=("parallel",)),
    )(emb, emba, thr, c3)
    return out
